# Initial kernel scaffold; baseline (speedup 1.0000x reference)
import jax
import jax.numpy as jnp
from jax.experimental import pallas as pl


def kernel(blocks, node_feats, edge_feats, W_self, W_neigh, b):
    raise NotImplementedError("write your pallas kernel here")


# speedup vs baseline: 14.4547x; 14.4547x over previous
import functools

import jax
import jax.numpy as jnp
from jax import lax
from jax.experimental import pallas as pl
from jax.experimental.pallas import tpu as pltpu
from jax.experimental.pallas import tpu_sc as plsc


_NUM_CORES = 2
_NUM_SUBCORES = 16
_LANES = 16


@functools.lru_cache(maxsize=None)
def _build_adj_fn(n_nodes: int, n_edges: int):
    n_workers = _NUM_CORES * _NUM_SUBCORES
    rows_per_worker = n_nodes // n_workers
    n_passes = 2
    rows = rows_per_worker // n_passes
    ech = 8192
    n_chunks = n_edges // ech

    mesh = plsc.VectorSubcoreMesh(core_axis_name="c", subcore_axis_name="s")

    @functools.partial(
        pl.kernel,
        mesh=mesh,
        compiler_params=pltpu.CompilerParams(needs_layout_passes=False),
        out_type=jax.ShapeDtypeStruct((n_nodes * n_nodes,), jnp.float32),
        scratch_types=[
            pltpu.VMEM(((rows + 1) * n_nodes,), jnp.float32),
            pltpu.VMEM((ech,), jnp.int32),
            pltpu.VMEM((ech,), jnp.int32),
        ],
    )
    def build_adj(src_hbm, dst_hbm, a_hbm, acc, srcb, dstb):
        wid = lax.axis_index("s") * _NUM_CORES + lax.axis_index("c")
        ones = jnp.full((_LANES,), 1.0, dtype=jnp.float32)
        zeros = jnp.zeros((_LANES,), dtype=jnp.float32)

        for p in range(n_passes):
            row0 = wid * rows_per_worker + p * rows

            def zero_body(i, _):
                acc[pl.ds(i * _LANES, _LANES)] = zeros
                return 0

            lax.fori_loop(0, (rows + 1) * n_nodes // _LANES, zero_body, 0)

            def chunk_body(c, _):
                pltpu.sync_copy(src_hbm.at[pl.ds(c * ech, ech)], srcb)
                pltpu.sync_copy(dst_hbm.at[pl.ds(c * ech, ech)], dstb)

                def edge_body(j, _):
                    sv = srcb[pl.ds(j * _LANES, _LANES)]
                    dv = dstb[pl.ds(j * _LANES, _LANES)]
                    rel = dv - row0
                    inr = (rel >= 0) & (rel < rows)
                    row = jnp.where(inr, rel, rows)
                    flat = row * n_nodes + sv
                    plsc.addupdate_scatter(acc, [flat], ones)
                    return 0

                lax.fori_loop(0, ech // _LANES, edge_body, 0)
                return 0

            lax.fori_loop(0, n_chunks, chunk_body, 0)
            pltpu.sync_copy(acc.at[pl.ds(0, rows * n_nodes)],
                            a_hbm.at[pl.ds(row0 * n_nodes, rows * n_nodes)])

    return build_adj


def _proj_kernel(y_ref, w_ref, yw_ref, ys_ref):
    o = lax.dot_general(y_ref[...], w_ref[...], (((1,), (0,)), ((), ())),
                        preferred_element_type=jnp.float32)
    c = yw_ref.shape[1]
    yw_ref[...] = o[:, :c]
    ys_ref[...] = o[:, c:]


def _agg_kernel(a_ref, yw_ref, ys_ref, bias_ref, g_ref, gt_ref, out_ref,
                acc_ref, deg_ref):
    k = pl.program_id(1)
    nk = pl.num_programs(1)

    @pl.when(k == 0)
    def _init():
        acc_ref[...] = jnp.zeros_like(acc_ref)
        deg_ref[...] = jnp.zeros_like(deg_ref)

    a = a_ref[...]
    acc_ref[...] += lax.dot_general(a, yw_ref[...], (((1,), (0,)), ((), ())),
                                    preferred_element_type=jnp.float32)
    deg_ref[...] += jnp.sum(a, axis=1, keepdims=True)

    @pl.when(k == nk - 1)
    def _epilogue():
        invd = 1.0 / jnp.maximum(deg_ref[...], 1.0)
        h = ys_ref[...] + acc_ref[...] * invd + bias_ref[...]
        hh = h * h
        n2 = lax.dot_general(hh, g_ref[...], (((1,), (0,)), ((), ())),
                             preferred_element_type=jnp.float32)
        r = 1.0 / jnp.maximum(jnp.sqrt(n2), 1e-12)
        d = lax.dot_general(r, gt_ref[...], (((1,), (0,)), ((), ())),
                            preferred_element_type=jnp.float32)
        out_ref[...] = jnp.maximum(h * d, 0.0)


def kernel(blocks, node_feats, edge_feats, W_self, W_neigh, b):
    del edge_feats
    bn, nn, tn, cin = node_feats.shape
    cout = W_self.shape[1]
    en = blocks.shape[1]
    g = bn * tn

    src = blocks[0].astype(jnp.int32)
    dst = blocks[1].astype(jnp.int32)
    adj = _build_adj_fn(nn, en)(src, dst).reshape(nn, nn)

    y = jnp.transpose(node_feats, (1, 0, 2, 3)).reshape(nn * g, cin)
    w_cat = jnp.concatenate([W_neigh, W_self], axis=1)

    rt = 8192
    yw, ys = pl.pallas_call(
        _proj_kernel,
        grid=(nn * g // rt,),
        in_specs=[
            pl.BlockSpec((rt, cin), lambda i: (i, 0)),
            pl.BlockSpec((cin, 2 * cout), lambda i: (0, 0)),
        ],
        out_specs=[
            pl.BlockSpec((rt, cout), lambda i: (i, 0)),
            pl.BlockSpec((rt, cout), lambda i: (i, 0)),
        ],
        out_shape=[jax.ShapeDtypeStruct((nn * g, cout), jnp.float32)] * 2,
    )(y, w_cat)

    gc = g * cout
    yw = yw.reshape(nn, gc)
    ys = ys.reshape(nn, gc)
    bias_row = jnp.tile(b, g)[None, :]
    gmat = jnp.repeat(jnp.eye(g, dtype=jnp.float32), cout, axis=0)

    bm, bk = 256, 256
    h = pl.pallas_call(
        _agg_kernel,
        grid=(nn // bm, nn // bk),
        in_specs=[
            pl.BlockSpec((bm, bk), lambda i, k: (i, k)),
            pl.BlockSpec((bk, gc), lambda i, k: (k, 0)),
            pl.BlockSpec((bm, gc), lambda i, k: (i, 0)),
            pl.BlockSpec((1, gc), lambda i, k: (0, 0)),
            pl.BlockSpec((gc, g), lambda i, k: (0, 0)),
            pl.BlockSpec((g, gc), lambda i, k: (0, 0)),
        ],
        out_specs=pl.BlockSpec((bm, gc), lambda i, k: (i, 0)),
        out_shape=jax.ShapeDtypeStruct((nn, gc), jnp.float32),
        scratch_shapes=[
            pltpu.VMEM((bm, gc), jnp.float32),
            pltpu.VMEM((bm, 1), jnp.float32),
        ],
    )(adj, yw, ys, bias_row, gmat, gmat.T)

    return h.reshape(nn, bn, tn, cout).transpose(1, 0, 2, 3)


# speedup vs baseline: 24.0731x; 1.6654x over previous
import functools

import jax
import jax.numpy as jnp
from jax import lax
from jax.experimental import pallas as pl
from jax.experimental.pallas import tpu as pltpu
from jax.experimental.pallas import tpu_sc as plsc


_NUM_CORES = 2
_NUM_SUBCORES = 16
_LANES = 16


@functools.lru_cache(maxsize=None)
def _build_adj_fn(n_nodes: int, n_edges: int):
    n_workers = _NUM_CORES * _NUM_SUBCORES
    rows_per_worker = n_nodes // n_workers
    n_passes = 2
    rows = rows_per_worker // n_passes
    ech = 16384
    n_chunks = n_edges // ech
    zunroll = 8
    eunroll = 4

    mesh = plsc.VectorSubcoreMesh(core_axis_name="c", subcore_axis_name="s")

    @functools.partial(
        pl.kernel,
        mesh=mesh,
        compiler_params=pltpu.CompilerParams(needs_layout_passes=False),
        out_type=jax.ShapeDtypeStruct((n_nodes * n_nodes,), jnp.float32),
        scratch_types=[
            pltpu.VMEM(((rows + 1) * n_nodes,), jnp.float32),
            pltpu.VMEM((ech,), jnp.int32),
            pltpu.VMEM((ech,), jnp.int32),
        ],
    )
    def build_adj(src_hbm, dst_hbm, a_hbm, acc, srcb, dstb):
        wid = lax.axis_index("s") * _NUM_CORES + lax.axis_index("c")
        ones = jnp.full((_LANES,), 1.0, dtype=jnp.float32)
        zeros = jnp.zeros((_LANES,), dtype=jnp.float32)

        for p in range(n_passes):
            row0 = wid * rows_per_worker + p * rows

            def zero_body(i, _):
                for u in range(zunroll):
                    acc[pl.ds((i * zunroll + u) * _LANES, _LANES)] = zeros
                return 0

            lax.fori_loop(0, rows * n_nodes // (_LANES * zunroll), zero_body, 0)

            def chunk_body(c, _):
                pltpu.sync_copy(src_hbm.at[pl.ds(c * ech, ech)], srcb)
                pltpu.sync_copy(dst_hbm.at[pl.ds(c * ech, ech)], dstb)

                def edge_body(j, _):
                    for u in range(eunroll):
                        o = (j * eunroll + u) * _LANES
                        sv = srcb[pl.ds(o, _LANES)]
                        dv = dstb[pl.ds(o, _LANES)]
                        rel = dv - row0
                        inr = rel.astype(jnp.uint32) < jnp.uint32(rows)
                        row = jnp.where(inr, rel, rows)
                        flat = row * n_nodes + sv
                        plsc.addupdate_scatter(acc, [flat], ones)
                    return 0

                lax.fori_loop(0, ech // (_LANES * eunroll), edge_body, 0)
                return 0

            lax.fori_loop(0, n_chunks, chunk_body, 0)
            pltpu.sync_copy(acc.at[pl.ds(0, rows * n_nodes)],
                            a_hbm.at[pl.ds(row0 * n_nodes, rows * n_nodes)])

    return build_adj


def _agg_kernel(a_ref, yk_ref, yi_ref, w2_ref, bias_ref, out_ref,
                acc_ref, deg_ref):
    k = pl.program_id(1)
    nk = pl.num_programs(1)

    @pl.when(k == 0)
    def _init():
        acc_ref[...] = jnp.zeros_like(acc_ref)
        deg_ref[...] = jnp.zeros_like(deg_ref)

    a = a_ref[...]
    acc_ref[...] += lax.dot_general(a.astype(jnp.bfloat16), yk_ref[...],
                                    (((1,), (0,)), ((), ())),
                                    preferred_element_type=jnp.float32)
    deg_ref[...] += jnp.sum(a, axis=1, keepdims=True)

    @pl.when(k == nk - 1)
    def _epilogue():
        invd = 1.0 / jnp.maximum(deg_ref[...], 1.0)
        w2 = w2_ref[...].astype(jnp.bfloat16)
        bias = bias_ref[...]
        yi = yi_ref[...]
        cc = w2_ref.shape[1]
        ng = yi_ref.shape[1] // cc
        for gi in range(ng):
            sl = pl.ds(gi * cc, cc)
            yg = yi[:, gi * cc:(gi + 1) * cc]
            sg = (acc_ref[:, sl] * invd).astype(jnp.bfloat16)
            cat = jnp.concatenate([yg, sg], axis=1)
            hg = lax.dot_general(cat, w2, (((1,), (0,)), ((), ())),
                                 preferred_element_type=jnp.float32) + bias
            n2 = jnp.sum(hg * hg, axis=1, keepdims=True)
            r = 1.0 / jnp.maximum(jnp.sqrt(n2), 1e-12)
            out_ref[:, sl] = jnp.maximum(hg * r, 0.0)


def kernel(blocks, node_feats, edge_feats, W_self, W_neigh, b):
    del edge_feats
    bn, nn, tn, cin = node_feats.shape
    cout = W_self.shape[1]
    en = blocks.shape[1]
    g = bn * tn
    gc = g * cout

    src = blocks[0].astype(jnp.int32)
    dst = blocks[1].astype(jnp.int32)
    adj = _build_adj_fn(nn, en)(src, dst).reshape(nn, nn)

    y = jnp.transpose(node_feats, (1, 0, 2, 3)).reshape(nn, gc)
    y = y.astype(jnp.bfloat16)
    w2 = jnp.concatenate([W_self, W_neigh], axis=0)

    bm, bk = 512, 512
    h = pl.pallas_call(
        _agg_kernel,
        grid=(nn // bm, nn // bk),
        in_specs=[
            pl.BlockSpec((bm, bk), lambda i, k: (i, k)),
            pl.BlockSpec((bk, gc), lambda i, k: (k, 0)),
            pl.BlockSpec((bm, gc), lambda i, k: (i, 0)),
            pl.BlockSpec((2 * cin, cout), lambda i, k: (0, 0)),
            pl.BlockSpec((1, cout), lambda i, k: (0, 0)),
        ],
        out_specs=pl.BlockSpec((bm, gc), lambda i, k: (i, 0)),
        out_shape=jax.ShapeDtypeStruct((nn, gc), jnp.float32),
        scratch_shapes=[
            pltpu.VMEM((bm, gc), jnp.float32),
            pltpu.VMEM((bm, 1), jnp.float32),
        ],
    )(adj, y, y, w2, b[None, :])

    return h.reshape(nn, bn, tn, cout).transpose(1, 0, 2, 3)


# speedup vs baseline: 25.4636x; 1.0578x over previous
import functools

import jax
import jax.numpy as jnp
import numpy as np
from jax import lax
from jax.experimental import pallas as pl
from jax.experimental.pallas import tpu as pltpu
from jax.experimental.pallas import tpu_sc as plsc


_NUM_CORES = 2
_NUM_SUBCORES = 16
_LANES = 16


@functools.lru_cache(maxsize=None)
def _build_adj_fn(n_nodes: int, n_edges: int):
    n_workers = _NUM_CORES * _NUM_SUBCORES
    rows_per_worker = n_nodes // n_workers
    n_passes = 2
    rows = rows_per_worker // n_passes
    ech = 16384
    n_chunks = n_edges // ech
    zunroll = 8
    eunroll = 4

    mesh = plsc.VectorSubcoreMesh(core_axis_name="c", subcore_axis_name="s")

    @functools.partial(
        pl.kernel,
        mesh=mesh,
        compiler_params=pltpu.CompilerParams(needs_layout_passes=False),
        out_type=jax.ShapeDtypeStruct((n_nodes, n_nodes), jnp.float32),
        scratch_types=[
            pltpu.VMEM((rows + 1, n_nodes), jnp.float32),
            pltpu.VMEM((ech,), jnp.int32),
            pltpu.VMEM((ech,), jnp.int32),
        ],
    )
    def build_adj(src_hbm, dst_hbm, a_hbm, acc, srcb, dstb):
        wid = lax.axis_index("s") * _NUM_CORES + lax.axis_index("c")
        ones = jnp.full((_LANES,), 1.0, dtype=jnp.float32)
        zeros = jnp.zeros((_LANES,), dtype=jnp.float32)

        for p in range(n_passes):
            row0 = wid * rows_per_worker + p * rows

            for r in range(rows):
                def zrow(i, _, r=r):
                    for u in range(zunroll):
                        acc[r, pl.ds((i * zunroll + u) * _LANES, _LANES)] = zeros
                    return 0
                lax.fori_loop(0, n_nodes // (_LANES * zunroll), zrow, 0)

            def chunk_body(c, _):
                pltpu.sync_copy(src_hbm.at[pl.ds(c * ech, ech)], srcb)
                pltpu.sync_copy(dst_hbm.at[pl.ds(c * ech, ech)], dstb)

                def edge_body(j, _):
                    for u in range(eunroll):
                        o = (j * eunroll + u) * _LANES
                        sv = srcb[pl.ds(o, _LANES)]
                        dv = dstb[pl.ds(o, _LANES)]
                        rel = dv - row0
                        inr = rel.astype(jnp.uint32) < jnp.uint32(rows)
                        row = jnp.where(inr, rel, rows)
                        plsc.addupdate_scatter(acc, [row, sv], ones)
                    return 0

                lax.fori_loop(0, ech // (_LANES * eunroll), edge_body, 0)
                return 0

            lax.fori_loop(0, n_chunks, chunk_body, 0)
            pltpu.sync_copy(acc.at[pl.ds(0, rows)],
                            a_hbm.at[pl.ds(row0, rows)])

    return build_adj


def _agg_kernel(a_ref, x_ref, ws_ref, wn_ref, bias_ref, g4_ref, out_ref,
                ab_ref, invd_ref):
    t = pl.program_id(0)
    bb = x_ref.shape[0]
    cc = x_ref.shape[3]

    @pl.when(t == 0)
    def _prep():
        ab = a_ref[...].astype(jnp.bfloat16)
        ab_ref[...] = ab
        deg = lax.dot_general(
            ab, jnp.ones((ab.shape[1], 8), jnp.bfloat16),
            (((1,), (0,)), ((), ())), preferred_element_type=jnp.float32)
        invd_ref[...] = 1.0 / jnp.maximum(deg[:, :1], 1.0)

    x = x_ref[...]
    cat = jnp.concatenate([x[b, 0] for b in range(bb)], axis=1)
    catb = cat.astype(jnp.bfloat16)
    agg = lax.dot_general(ab_ref[...], catb, (((1,), (0,)), ((), ())),
                          preferred_element_type=jnp.float32)
    s = (agg * invd_ref[...]).astype(jnp.bfloat16)
    hs = lax.dot_general(catb, ws_ref[...], (((1,), (0,)), ((), ())),
                         preferred_element_type=jnp.float32)
    hn = lax.dot_general(s, wn_ref[...], (((1,), (0,)), ((), ())),
                         preferred_element_type=jnp.float32)
    h = hs + hn + bias_ref[...]
    n2 = lax.dot_general(h * h, g4_ref[...], (((1,), (0,)), ((), ())),
                         preferred_element_type=jnp.float32)
    r = 1.0 / jnp.maximum(jnp.sqrt(n2), 1e-12)
    for b in range(bb):
        out_ref[b, 0] = jnp.maximum(h[:, b * cc:(b + 1) * cc] * r[:, b:b + 1],
                                    0.0)


def kernel(blocks, node_feats, edge_feats, W_self, W_neigh, b):
    del edge_feats
    bn, nn, tn, cin = node_feats.shape
    cout = W_self.shape[1]
    en = blocks.shape[1]
    bc = bn * cout

    src = blocks[0].astype(jnp.int32)
    dst = blocks[1].astype(jnp.int32)
    adj = _build_adj_fn(nn, en)(src, dst)

    x = jnp.transpose(node_feats, (0, 2, 1, 3))
    eye_b = jnp.eye(bn, dtype=jnp.float32)
    ws4 = jnp.kron(eye_b, W_self).astype(jnp.bfloat16)
    wn4 = jnp.kron(eye_b, W_neigh).astype(jnp.bfloat16)
    g4 = jnp.kron(eye_b, jnp.ones((cout, 1), jnp.float32))
    bias_row = jnp.tile(b, bn)[None, :]

    h = pl.pallas_call(
        _agg_kernel,
        grid=(tn,),
        in_specs=[
            pl.BlockSpec((nn, nn), lambda t: (0, 0)),
            pl.BlockSpec((bn, 1, nn, cin), lambda t: (0, t, 0, 0)),
            pl.BlockSpec((bc, bc), lambda t: (0, 0)),
            pl.BlockSpec((bc, bc), lambda t: (0, 0)),
            pl.BlockSpec((1, bc), lambda t: (0, 0)),
            pl.BlockSpec((bc, bn), lambda t: (0, 0)),
        ],
        out_specs=pl.BlockSpec((bn, 1, nn, cout), lambda t: (0, t, 0, 0)),
        out_shape=jax.ShapeDtypeStruct((bn, tn, nn, cout), jnp.float32),
        scratch_shapes=[
            pltpu.VMEM((nn, nn), jnp.bfloat16),
            pltpu.VMEM((nn, 1), jnp.float32),
        ],
    )(adj, x, ws4, wn4, bias_row, g4)

    return jnp.transpose(h, (0, 2, 1, 3))


# speedup vs baseline: 26.8795x; 1.0556x over previous
import functools

import jax
import jax.numpy as jnp
import numpy as np
from jax import lax
from jax.experimental import pallas as pl
from jax.experimental.pallas import tpu as pltpu
from jax.experimental.pallas import tpu_sc as plsc


_NUM_CORES = 2
_NUM_SUBCORES = 16
_LANES = 16


@functools.lru_cache(maxsize=None)
def _build_adj_fn(n_nodes: int, n_edges: int):
    n_workers = _NUM_CORES * _NUM_SUBCORES
    rows_per_worker = n_nodes // n_workers
    n_passes = 2
    rows = rows_per_worker // n_passes
    ech = 16384
    n_chunks = n_edges // ech
    zunroll = 8
    eunroll = 4

    mesh = plsc.VectorSubcoreMesh(core_axis_name="c", subcore_axis_name="s")

    @functools.partial(
        pl.kernel,
        mesh=mesh,
        compiler_params=pltpu.CompilerParams(needs_layout_passes=False),
        out_type=jax.ShapeDtypeStruct((n_nodes, n_nodes), jnp.float32),
        scratch_types=[
            pltpu.VMEM((rows + 1, n_nodes), jnp.float32),
            pltpu.VMEM((ech,), jnp.int32),
            pltpu.VMEM((ech,), jnp.int32),
        ],
    )
    def build_adj(src_hbm, dst_hbm, a_hbm, acc, srcb, dstb):
        wid = lax.axis_index("s") * _NUM_CORES + lax.axis_index("c")
        ones = jnp.full((_LANES,), 1.0, dtype=jnp.float32)
        zeros = jnp.zeros((_LANES,), dtype=jnp.float32)

        for p in range(n_passes):
            row0 = wid * rows_per_worker + p * rows

            for r in range(rows):
                def zrow(i, _, r=r):
                    for u in range(zunroll):
                        acc[r, pl.ds((i * zunroll + u) * _LANES, _LANES)] = zeros
                    return 0
                lax.fori_loop(0, n_nodes // (_LANES * zunroll), zrow, 0)

            def chunk_body(c, _):
                pltpu.sync_copy(src_hbm.at[pl.ds(c * ech, ech)], srcb)
                pltpu.sync_copy(dst_hbm.at[pl.ds(c * ech, ech)], dstb)

                def edge_body(j, _):
                    for u in range(eunroll):
                        o = (j * eunroll + u) * _LANES
                        sv = srcb[pl.ds(o, _LANES)]
                        dv = dstb[pl.ds(o, _LANES)]
                        rel = dv - row0
                        inr = rel.astype(jnp.uint32) < jnp.uint32(rows)
                        row = jnp.where(inr, rel, rows)
                        plsc.addupdate_scatter(acc, [row, sv], ones)
                    return 0

                lax.fori_loop(0, ech // (_LANES * eunroll), edge_body, 0)
                return 0

            lax.fori_loop(0, n_chunks, chunk_body, 0)
            pltpu.sync_copy(acc.at[pl.ds(0, rows)],
                            a_hbm.at[pl.ds(row0, rows)])

    return build_adj


def _agg_kernel(a_ref, x_ref, ws_ref, wn_ref, bias_ref, g4_ref, g4t_ref,
                out_ref, ab_ref, invd_ref):
    t = pl.program_id(0)

    @pl.when(t == 0)
    def _prep():
        ab = a_ref[...].astype(jnp.bfloat16)
        ab_ref[...] = ab
        deg = lax.dot_general(
            ab, jnp.ones((ab.shape[1], 8), jnp.bfloat16),
            (((1,), (0,)), ((), ())), preferred_element_type=jnp.float32)
        invd_ref[...] = 1.0 / jnp.maximum(deg[:, :1], 1.0)

    x = x_ref[0]
    agg = lax.dot_general(ab_ref[...], x, (((1,), (0,)), ((), ())),
                          preferred_element_type=jnp.float32)
    s = (agg * invd_ref[...]).astype(jnp.bfloat16)
    hs = lax.dot_general(x, ws_ref[...], (((1,), (0,)), ((), ())),
                         preferred_element_type=jnp.float32)
    hn = lax.dot_general(s, wn_ref[...], (((1,), (0,)), ((), ())),
                         preferred_element_type=jnp.float32)
    h = hs + hn + bias_ref[...]
    n2 = lax.dot_general(h * h, g4_ref[...], (((1,), (0,)), ((), ())),
                         preferred_element_type=jnp.float32)
    r = 1.0 / jnp.maximum(jnp.sqrt(n2), 1e-12)
    d = lax.dot_general(r, g4t_ref[...], (((1,), (0,)), ((), ())),
                        preferred_element_type=jnp.float32)
    out_ref[0] = jnp.maximum(h * d, 0.0)


def kernel(blocks, node_feats, edge_feats, W_self, W_neigh, b):
    del edge_feats
    bn, nn, tn, cin = node_feats.shape
    cout = W_self.shape[1]
    en = blocks.shape[1]
    bc = bn * cout

    src = blocks[0].astype(jnp.int32)
    dst = blocks[1].astype(jnp.int32)
    adj = _build_adj_fn(nn, en)(src, dst)

    x = jnp.transpose(node_feats, (2, 1, 0, 3)).reshape(tn, nn, bc)
    x = x.astype(jnp.bfloat16)
    eye_b = jnp.eye(bn, dtype=jnp.float32)
    ws4 = jnp.kron(eye_b, W_self).astype(jnp.bfloat16)
    wn4 = jnp.kron(eye_b, W_neigh).astype(jnp.bfloat16)
    g4 = jnp.kron(eye_b, jnp.ones((cout, 1), jnp.float32))
    g4t = jnp.kron(eye_b, jnp.ones((1, cout), jnp.float32))
    bias_row = jnp.tile(b, bn)[None, :]

    h = pl.pallas_call(
        _agg_kernel,
        grid=(tn,),
        in_specs=[
            pl.BlockSpec((nn, nn), lambda t: (0, 0)),
            pl.BlockSpec((1, nn, bc), lambda t: (t, 0, 0)),
            pl.BlockSpec((bc, bc), lambda t: (0, 0)),
            pl.BlockSpec((bc, bc), lambda t: (0, 0)),
            pl.BlockSpec((1, bc), lambda t: (0, 0)),
            pl.BlockSpec((bc, bn), lambda t: (0, 0)),
            pl.BlockSpec((bn, bc), lambda t: (0, 0)),
        ],
        out_specs=pl.BlockSpec((1, nn, bc), lambda t: (t, 0, 0)),
        out_shape=jax.ShapeDtypeStruct((tn, nn, bc), jnp.float32),
        scratch_shapes=[
            pltpu.VMEM((nn, nn), jnp.bfloat16),
            pltpu.VMEM((nn, 1), jnp.float32),
        ],
    )(adj, x, ws4, wn4, bias_row, g4, g4t)

    return jnp.transpose(h.reshape(tn, nn, bn, cout), (2, 1, 0, 3))
